# Initial kernel scaffold; baseline (speedup 1.0000x reference)
import jax
import jax.numpy as jnp
from jax.experimental import pallas as pl


def kernel(x, edge_index, W_self0, W_neigh0, b0, W_self1, W_neigh1, b1):
    raise NotImplementedError("write your pallas kernel here")


# speedup vs baseline: 8.5719x; 8.5719x over previous
import functools

import jax
import jax.numpy as jnp
from jax import lax
from jax.experimental import pallas as pl
from jax.experimental.pallas import tpu as pltpu
from jax.experimental.pallas import tpu_sc as plsc

_NC = 2
_NS = 16
_K = 128


def _unpack_chunk(pk, c, half, idxbuf):
  for g in range(_K // 32):
    w = pk[c, pl.ds(half * (_K // 2) + 16 * g, 16)]
    idxbuf[0, pl.ds(32 * g, 16)] = w & 0xFFFF
    idxbuf[0, pl.ds(32 * g + 16, 16)] = lax.shift_right_logical(w, 16)


def _splits(n):
  wr = (n // _NS) & ~7
  return wr, n - _NS * wr


def _make_agg(N, D, CHUNKS):
  NPAD = N + 16
  ZROWS = NPAD // _NS
  WR, REM = _splits(N)
  mesh = plsc.VectorSubcoreMesh(core_axis_name="c", subcore_axis_name="s")

  @functools.partial(
      pl.kernel, mesh=mesh,
      out_type=[jax.ShapeDtypeStruct((_NC, N, D), jnp.float32)],
      scratch_types=[
          pltpu.VMEM((CHUNKS, _K), jnp.int32),
          pltpu.VMEM((CHUNKS // 2, _K), jnp.int32),
          pltpu.VMEM((1, _K), jnp.int32),
          pltpu.VMEM((_K, D), jnp.float32),
          pltpu.VMEM((_K, D), jnp.float32),
          pltpu.VMEM_SHARED((NPAD, D), jnp.float32),
          pltpu.SemaphoreType.DMA,
          pltpu.SemaphoreType.DMA,
      ])
  def agg(x_hbm, src_hbm, dstp_hbm, out_hbm,
          srcv, dstv, idxbuf, buf0, buf1, acc, sem0, sem1):
    c = lax.axis_index("c")
    s = lax.axis_index("s")
    w = c * _NS + s

    pltpu.sync_copy(src_hbm.at[w], srcv)
    pltpu.sync_copy(dstp_hbm.at[w], dstv)

    zero16 = jnp.zeros((16,), jnp.float32)

    def fill_zero(r, carry):
      for j in range(D // 16):
        buf0[r, pl.ds(j * 16, 16)] = zero16
      return carry

    lax.fori_loop(0, _K, fill_zero, 0)
    for off in range(0, ZROWS, _K):
      sz = min(_K, ZROWS - off)
      pltpu.sync_copy(buf0.at[pl.ds(0, sz)],
                      acc.at[pl.ds(s * ZROWS + off, sz)])
    plsc.subcore_barrier()

    pltpu.async_copy(x_hbm.at[srcv.at[0]], buf0, sem0)
    pltpu.async_copy(x_hbm.at[srcv.at[1]], buf1, sem1)

    def step(i, carry):
      c0 = 2 * i

      pltpu.make_async_copy(x_hbm.at[srcv.at[c0]], buf0, sem0).wait()
      _unpack_chunk(dstv, i, 0, idxbuf)
      pltpu.sync_copy(buf0, acc.at[idxbuf.at[0]], add=True)

      @pl.when(c0 + 2 < CHUNKS)
      def _():
        pltpu.async_copy(x_hbm.at[srcv.at[c0 + 2]], buf0, sem0)

      pltpu.make_async_copy(x_hbm.at[srcv.at[c0 + 1]], buf1, sem1).wait()
      _unpack_chunk(dstv, i, 1, idxbuf)
      pltpu.sync_copy(buf1, acc.at[idxbuf.at[0]], add=True)

      @pl.when(c0 + 3 < CHUNKS)
      def _():
        pltpu.async_copy(x_hbm.at[srcv.at[c0 + 3]], buf1, sem1)

      return carry

    lax.fori_loop(0, CHUNKS // 2, step, 0)
    plsc.subcore_barrier()

    pltpu.sync_copy(acc.at[pl.ds(s * WR, WR)],
                    out_hbm.at[c, pl.ds(s * WR, WR)])
    if REM:
      @pl.when(s == 0)
      def _():
        pltpu.sync_copy(acc.at[pl.ds(_NS * WR, REM)],
                        out_hbm.at[c, pl.ds(_NS * WR, REM)])

  return agg


def _make_tc_layer(N, D, relu):
  BN = 2000
  grid = (N // BN,)

  def body(h_ref, p0_ref, p1_ref, d0_ref, d1_ref, ws_ref, wn_ref, b_ref,
           o_ref):
    deg = jnp.maximum(d0_ref[:, 0:1] + d1_ref[:, 0:1], 1.0)
    hn = (p0_ref[...] + p1_ref[...]) / deg
    o = (jnp.dot(h_ref[...], ws_ref[...], preferred_element_type=jnp.float32)
         + jnp.dot(hn, wn_ref[...], preferred_element_type=jnp.float32)
         + b_ref[...])
    if relu:
      o = jnp.maximum(o, 0.0)
    o_ref[...] = o

  row_spec = pl.BlockSpec((BN, D), lambda i: (i, 0))
  deg_spec = pl.BlockSpec((BN, 16), lambda i: (i, 0))
  full_spec = pl.BlockSpec((D, D), lambda i: (0, 0))
  bias_spec = pl.BlockSpec((1, D), lambda i: (0, 0))

  return pl.pallas_call(
      body,
      grid=grid,
      in_specs=[row_spec, row_spec, row_spec, deg_spec, deg_spec,
                full_spec, full_spec, bias_spec],
      out_specs=row_spec,
      out_shape=jax.ShapeDtypeStruct((N, D), jnp.float32),
  )


def kernel(x, edge_index, W_self0, W_neigh0, b0, W_self1, W_neigh1, b1):
  N, D = x.shape
  E = edge_index.shape[1]
  NW = _NC * _NS

  per_w = -(-E // NW)
  chunks = -(-per_w // _K)
  chunks += chunks % 2
  e_pad = NW * chunks * _K
  pad = e_pad - E
  src = edge_index[0]
  dst = edge_index[1]
  if pad:
    pad_src = (jnp.arange(pad, dtype=jnp.int32) * 97) % N
    pad_dst = N + (jnp.arange(pad, dtype=jnp.int32) % 16)
    src = jnp.concatenate([src, pad_src])
    dst = jnp.concatenate([dst, pad_dst])
  src = src.reshape(NW, chunks, _K)
  d = dst.reshape(NW, chunks, _K // 32, 2, 16)
  dstp = (d[:, :, :, 0, :] | (d[:, :, :, 1, :] << 16)).reshape(
      NW, chunks // 2, _K)

  agg = _make_agg(N, D, chunks)
  layer0 = _make_tc_layer(N, D, relu=True)
  layer1 = _make_tc_layer(N, D, relu=False)

  (deg128,) = agg(jnp.ones((N, D), jnp.float32), src, dstp)
  deg = deg128[:, :, :16]
  (p,) = agg(x, src, dstp)
  h1 = layer0(x, p[0], p[1], deg[0], deg[1], W_self0, W_neigh0,
              b0.reshape(1, D))
  (q,) = agg(h1, src, dstp)
  out = layer1(h1, q[0], q[1], deg[0], deg[1], W_self1, W_neigh1,
               b1.reshape(1, D))
  return out


# speedup vs baseline: 9.8015x; 1.1434x over previous
import functools

import jax
import jax.numpy as jnp
from jax import lax
from jax.experimental import pallas as pl
from jax.experimental.pallas import tpu as pltpu
from jax.experimental.pallas import tpu_sc as plsc

_NC = 2
_NS = 16
_K = 128


def _unpack_chunk(pk, c, half, idxbuf):
  for g in range(_K // 32):
    w = pk[c, pl.ds(half * (_K // 2) + 16 * g, 16)]
    idxbuf[0, pl.ds(32 * g, 16)] = w & 0xFFFF
    idxbuf[0, pl.ds(32 * g + 16, 16)] = lax.shift_right_logical(w, 16)


def _splits(n):
  wr = (n // _NS) & ~7
  return wr, n - _NS * wr


def _make_agg(N, D, CHUNKS):
  NPAD = N + 16
  ZROWS = NPAD // _NS
  WR, REM = _splits(N)
  mesh = plsc.VectorSubcoreMesh(core_axis_name="c", subcore_axis_name="s")

  @functools.partial(
      pl.kernel, mesh=mesh,
      out_type=[jax.ShapeDtypeStruct((_NC, N, D), jnp.float32)],
      scratch_types=[
          pltpu.VMEM((CHUNKS, _K), jnp.int32),
          pltpu.VMEM((CHUNKS // 2, _K), jnp.int32),
          pltpu.VMEM((1, _K), jnp.int32),
          pltpu.VMEM((_K, D), jnp.float32),
          pltpu.VMEM((_K, D), jnp.float32),
          pltpu.VMEM_SHARED((NPAD, D), jnp.float32),
          pltpu.SemaphoreType.DMA,
          pltpu.SemaphoreType.DMA,
      ])
  def agg(x_hbm, src_hbm, dstp_hbm, out_hbm,
          srcv, dstv, idxbuf, buf0, buf1, acc, sem0, sem1):
    c = lax.axis_index("c")
    s = lax.axis_index("s")
    w = c * _NS + s

    pltpu.sync_copy(src_hbm.at[w], srcv)
    pltpu.sync_copy(dstp_hbm.at[w], dstv)

    zero16 = jnp.zeros((16,), jnp.float32)

    def fill_zero(r, carry):
      for j in range(D // 16):
        buf0[r, pl.ds(j * 16, 16)] = zero16
      return carry

    lax.fori_loop(0, _K, fill_zero, 0)
    for off in range(0, ZROWS, _K):
      sz = min(_K, ZROWS - off)
      pltpu.sync_copy(buf0.at[pl.ds(0, sz)],
                      acc.at[pl.ds(s * ZROWS + off, sz)])
    plsc.subcore_barrier()

    pltpu.async_copy(x_hbm.at[srcv.at[0]], buf0, sem0)
    pltpu.async_copy(x_hbm.at[srcv.at[1]], buf1, sem1)

    def step(i, carry):
      c0 = 2 * i

      pltpu.make_async_copy(x_hbm.at[srcv.at[c0]], buf0, sem0).wait()
      _unpack_chunk(dstv, i, 0, idxbuf)
      pltpu.sync_copy(buf0, acc.at[idxbuf.at[0]], add=True)

      @pl.when(c0 + 2 < CHUNKS)
      def _():
        pltpu.async_copy(x_hbm.at[srcv.at[c0 + 2]], buf0, sem0)

      pltpu.make_async_copy(x_hbm.at[srcv.at[c0 + 1]], buf1, sem1).wait()
      _unpack_chunk(dstv, i, 1, idxbuf)
      pltpu.sync_copy(buf1, acc.at[idxbuf.at[0]], add=True)

      @pl.when(c0 + 3 < CHUNKS)
      def _():
        pltpu.async_copy(x_hbm.at[srcv.at[c0 + 3]], buf1, sem1)

      return carry

    lax.fori_loop(0, CHUNKS // 2, step, 0)
    plsc.subcore_barrier()

    pltpu.sync_copy(acc.at[pl.ds(s * WR, WR)],
                    out_hbm.at[c, pl.ds(s * WR, WR)])
    if REM:
      @pl.when(s == 0)
      def _():
        pltpu.sync_copy(acc.at[pl.ds(_NS * WR, REM)],
                        out_hbm.at[c, pl.ds(_NS * WR, REM)])

  return agg


def _make_deg(N, D, CHUNKS):
  NPAD = N + 16
  ZROWS = NPAD // _NS
  WR, REM = _splits(N)
  mesh = plsc.VectorSubcoreMesh(core_axis_name="c", subcore_axis_name="s")

  @functools.partial(
      pl.kernel, mesh=mesh,
      out_type=[jax.ShapeDtypeStruct((_NC, N, D), jnp.float32)],
      scratch_types=[
          pltpu.VMEM((CHUNKS // 2, _K), jnp.int32),
          pltpu.VMEM((1, _K), jnp.int32),
          pltpu.VMEM((_K, D), jnp.float32),
          pltpu.VMEM_SHARED((NPAD, D), jnp.float32),
      ])
  def deg(dstp_hbm, out_hbm, dstv, idxbuf, onesv, dacc):
    c = lax.axis_index("c")
    s = lax.axis_index("s")
    w = c * _NS + s

    pltpu.sync_copy(dstp_hbm.at[w], dstv)

    zero16 = jnp.zeros((16,), jnp.float32)

    def fill_zero(r, carry):
      for j in range(D // 16):
        onesv[r, pl.ds(j * 16, 16)] = zero16
      return carry

    lax.fori_loop(0, _K, fill_zero, 0)
    for off in range(0, ZROWS, _K):
      sz = min(_K, ZROWS - off)
      pltpu.sync_copy(onesv.at[pl.ds(0, sz)],
                      dacc.at[pl.ds(s * ZROWS + off, sz)])

    one16 = jnp.ones((16,), jnp.float32)

    def fill_one(r, carry):
      for j in range(D // 16):
        onesv[r, pl.ds(j * 16, 16)] = one16
      return carry

    lax.fori_loop(0, _K, fill_one, 0)
    plsc.subcore_barrier()

    def step(i, carry):
      _unpack_chunk(dstv, i, 0, idxbuf)
      pltpu.sync_copy(onesv, dacc.at[idxbuf.at[0]], add=True)
      _unpack_chunk(dstv, i, 1, idxbuf)
      pltpu.sync_copy(onesv, dacc.at[idxbuf.at[0]], add=True)
      return carry

    lax.fori_loop(0, CHUNKS // 2, step, 0)
    plsc.subcore_barrier()

    pltpu.sync_copy(dacc.at[pl.ds(s * WR, WR)],
                    out_hbm.at[c, pl.ds(s * WR, WR)])
    if REM:
      @pl.when(s == 0)
      def _():
        pltpu.sync_copy(dacc.at[pl.ds(_NS * WR, REM)],
                        out_hbm.at[c, pl.ds(_NS * WR, REM)])

  return deg


def _make_tc_layer(N, D, relu):
  BN = 2000
  grid = (N // BN,)

  def body(h_ref, p_ref, deg_ref, ws_ref, wn_ref, b_ref, o_ref):
    deg = jnp.maximum(deg_ref[0, :, 0:1] + deg_ref[1, :, 0:1], 1.0)
    hn = (p_ref[0] + p_ref[1]) / deg
    o = (jnp.dot(h_ref[...], ws_ref[...], preferred_element_type=jnp.float32)
         + jnp.dot(hn, wn_ref[...], preferred_element_type=jnp.float32)
         + b_ref[...])
    if relu:
      o = jnp.maximum(o, 0.0)
    o_ref[...] = o

  row_spec = pl.BlockSpec((BN, D), lambda i: (i, 0))
  pair_spec = pl.BlockSpec((2, BN, D), lambda i: (0, i, 0))
  full_spec = pl.BlockSpec((D, D), lambda i: (0, 0))
  bias_spec = pl.BlockSpec((1, D), lambda i: (0, 0))

  return pl.pallas_call(
      body,
      grid=grid,
      in_specs=[row_spec, pair_spec, pair_spec,
                full_spec, full_spec, bias_spec],
      out_specs=row_spec,
      out_shape=jax.ShapeDtypeStruct((N, D), jnp.float32),
  )


def kernel(x, edge_index, W_self0, W_neigh0, b0, W_self1, W_neigh1, b1):
  N, D = x.shape
  E = edge_index.shape[1]
  NW = _NC * _NS

  per_w = -(-E // NW)
  chunks = -(-per_w // _K)
  chunks += chunks % 2
  e_pad = NW * chunks * _K
  pad = e_pad - E
  src = edge_index[0]
  dst = edge_index[1]
  if pad:
    pad_src = (jnp.arange(pad, dtype=jnp.int32) * 97) % N
    pad_dst = N + (jnp.arange(pad, dtype=jnp.int32) % 16)
    src = jnp.concatenate([src, pad_src])
    dst = jnp.concatenate([dst, pad_dst])
  src = src.reshape(NW, chunks, _K)
  d = dst.reshape(NW, chunks, _K // 32, 2, 16)
  dstp = (d[:, :, :, 0, :] | (d[:, :, :, 1, :] << 16)).reshape(
      NW, chunks // 2, _K)

  agg = _make_agg(N, D, chunks)
  degk = _make_deg(N, D, chunks)
  layer0 = _make_tc_layer(N, D, relu=True)
  layer1 = _make_tc_layer(N, D, relu=False)

  (deg128,) = degk(dstp)
  (p,) = agg(x, src, dstp)
  h1 = layer0(x, p, deg128, W_self0, W_neigh0, b0.reshape(1, D))
  (q,) = agg(h1, src, dstp)
  out = layer1(h1, q, deg128, W_self1, W_neigh1, b1.reshape(1, D))
  return out
